# baseline (speedup 1.0000x reference)
import functools

import jax
import jax.numpy as jnp
from jax import lax
from jax.experimental import pallas as pl
from jax.experimental.pallas import tpu as pltpu
from jax.experimental.pallas import tpu_sc as plsc

N = 32768
D = 256
NC = 2
NS = 16
NW = NC * NS
RPW = N // NW
CH = 64
NCH = RPW // CH
L = 16


def _sc_body(x_hbm, idx_hbm, tab_hbm, out_hbm, idx_v, xv, tv, semx, semt):
    wid = lax.axis_index("s") * NC + lax.axis_index("c")
    base = wid * RPW

    pltpu.sync_copy(idx_hbm.at[wid], idx_v)
    for ci in range(NCH):
        for j in range(CH // L):
            sl = (ci, pl.ds(j * L, L))
            idx_v[sl] = jnp.clip(idx_v[sl], 0, 16)

    def chunk(ci, _):
        rb = base + ci * CH
        cpx = pltpu.async_copy(x_hbm.at[pl.ds(rb, CH)], xv, semx)
        cpt = pltpu.async_copy(tab_hbm.at[idx_v.at[ci]], tv, semt)
        cpx.wait()
        cpt.wait()

        def row_add(r, _):
            for j in range(D // L):
                sl = (r, pl.ds(j * L, L))
                plsc.addupdate(xv.at[sl], tv[sl])
            return 0

        lax.fori_loop(0, CH, row_add, 0)
        pltpu.sync_copy(xv, out_hbm.at[pl.ds(rb, CH)])
        return 0

    lax.fori_loop(0, NCH, chunk, 0)


@jax.jit
def _sc_call(xr, idx3, table):
    mesh = plsc.VectorSubcoreMesh(core_axis_name="c", subcore_axis_name="s")
    f = functools.partial(
        pl.kernel,
        mesh=mesh,
        out_type=jax.ShapeDtypeStruct((N, D), jnp.float32),
        scratch_types=[
            pltpu.VMEM((NCH, CH), jnp.int32),
            pltpu.VMEM((CH, D), jnp.float32),
            pltpu.VMEM((CH, D), jnp.float32),
            pltpu.SemaphoreType.DMA,
            pltpu.SemaphoreType.DMA,
        ],
    )(_sc_body)
    return f(xr, idx3, table)


def kernel(x, in_chan_matrix, embed_table):
    B, S, Dd = x.shape
    xr = x.reshape(B * S, Dd)
    idx3 = in_chan_matrix.astype(jnp.int32).reshape(NW, NCH, CH)
    out = _sc_call(xr, idx3, embed_table)
    return out.reshape(B, S, Dd)
